# baseline (speedup 1.0000x reference)
import jax
import jax.numpy as jnp
from jax import lax
from jax.experimental import pallas as pl
from jax.experimental.pallas import tpu as pltpu
from jax.experimental.pallas import tpu_sc as plsc

D = 64
B = 4096
SEQ = 200
NT = 20
NG = SEQ - NT
NC = 2
NS = 16
NW = NC * NS
BPW = B // NW


def _sc_body(idxg_hbm, wte_hbm, learned_hbm, out_hbm,
             idx_v, learned_v, rows, tiles, sem_g, sem_w):
    c_ax = lax.axis_index("c")
    s_ax = lax.axis_index("s")
    bt = s_ax * NC + c_ax
    col0 = bt * BPW

    pltpu.sync_copy(idxg_hbm.at[:, pl.ds(col0, BPW)], idx_v)
    pltpu.sync_copy(learned_hbm, learned_v)

    iotas = [lax.iota(jnp.int32, 16) + c * 16 for c in range(8)]

    def start_g(j, slot):
        pltpu.async_copy(wte_hbm.at[idx_v.at[j]], rows.at[slot],
                         sem_g.at[slot])

    def wait_g(j, slot):
        pltpu.make_async_copy(wte_hbm.at[idx_v.at[j]], rows.at[slot],
                              sem_g.at[slot]).wait()

    def start_wb(s, slot):
        pltpu.async_copy(tiles.at[slot], out_hbm.at[s, :, bt],
                         sem_w.at[slot])

    def wait_wb(s, slot):
        pltpu.make_async_copy(tiles.at[slot], out_hbm.at[s, :, bt],
                              sem_w.at[slot]).wait()

    def build_learned(s, slot):
        def dbody(d, carry):
            splat = plsc.load_gather(
                learned_v,
                [jnp.full((16,), s, jnp.int32), jnp.full((16,), d, jnp.int32)])
            dt = d // 8
            dr = d % 8
            for c in range(8):
                tiles[slot, dt, dr, pl.ds(c * 16, 16)] = splat
            return carry
        lax.fori_loop(0, D, dbody, 0)

    def transpose_rows(slot):
        def dbody(d, carry):
            col = jnp.full((16,), d, jnp.int32)
            dt = d // 8
            dr = d % 8
            for c in range(8):
                v = plsc.load_gather(rows.at[slot], [iotas[c], col])
                tiles[slot, dt, dr, pl.ds(c * 16, 16)] = v
            return carry
        lax.fori_loop(0, D, dbody, 0)

    for s in range(NT):
        slot = s & 1
        if s >= 2:
            wait_wb(s - 2, slot)
        build_learned(s, slot)
        start_wb(s, slot)

    start_g(0, 0)

    def jbody(j, carry):
        slot = j & 1
        wait_g(j, slot)
        start_g(j + 1, 1 - slot)
        wait_wb(NT - 2 + j, slot)
        transpose_rows(slot)
        start_wb(NT + j, slot)
        return carry

    lax.fori_loop(0, NG - 1, jbody, 0)

    wait_g(NG - 1, 1)
    wait_wb(NT + NG - 3, 1)
    transpose_rows(1)
    start_wb(SEQ - 1, 1)
    wait_wb(SEQ - 2, 0)
    wait_wb(SEQ - 1, 1)


def kernel(tokens, wte_weight, learned_embedding):
    tokens = tokens.astype(jnp.int32)
    idxg = tokens[:, NT:].T

    mesh = plsc.VectorSubcoreMesh(core_axis_name="c", subcore_axis_name="s")
    run = pl.kernel(
        _sc_body,
        out_type=jax.ShapeDtypeStruct((SEQ, 8, NW, 8, BPW), jnp.float32),
        mesh=mesh,
        scratch_types=[
            pltpu.VMEM((NG, BPW), jnp.int32),
            pltpu.VMEM((NT, D), jnp.float32),
            pltpu.VMEM((2, BPW, D), jnp.float32),
            pltpu.VMEM((2, 8, 8, BPW), jnp.float32),
            pltpu.SemaphoreType.DMA((2,)),
            pltpu.SemaphoreType.DMA((2,)),
        ],
        compiler_params=pltpu.CompilerParams(
            use_tc_tiling_on_sc=False, needs_layout_passes=False),
    )
    x = run(idxg, wte_weight, learned_embedding)
    return x.transpose(2, 4, 0, 1, 3).reshape(B, SEQ, D)


# speedup vs baseline: 2.6626x; 2.6626x over previous
import jax
import jax.numpy as jnp
from jax import lax
from jax.experimental import pallas as pl
from jax.experimental.pallas import tpu as pltpu
from jax.experimental.pallas import tpu_sc as plsc

D = 64
B = 4096
SEQ = 200
NT = 20
CH = 96
NC = 2
NS = 16
NW = NC * NS
BPW = B // NW


def _sc_body(idx_hbm, wte_hbm, learned_hbm, out_hbm, idx_v, stage, sem_g, sem_w):
    c = lax.axis_index("c")
    s = lax.axis_index("s")
    wid = s * NC + c
    base = wid * BPW

    pltpu.sync_copy(idx_hbm.at[pl.ds(base, BPW)], idx_v)
    pltpu.sync_copy(learned_hbm, stage.at[0, pl.ds(0, NT)])
    pltpu.sync_copy(learned_hbm, stage.at[1, pl.ds(0, NT)])

    def start_gather(b, slot):
        pltpu.async_copy(
            wte_hbm.at[idx_v.at[b, 0]], stage.at[slot, pl.ds(NT, CH)],
            sem_g.at[slot])
        pltpu.async_copy(
            wte_hbm.at[idx_v.at[b, 1]], stage.at[slot, pl.ds(SEQ - CH, CH)],
            sem_g.at[slot])

    def wait_gather(b, slot):
        pltpu.make_async_copy(
            wte_hbm.at[idx_v.at[b, 0]], stage.at[slot, pl.ds(NT, CH)],
            sem_g.at[slot]).wait()
        pltpu.make_async_copy(
            wte_hbm.at[idx_v.at[b, 1]], stage.at[slot, pl.ds(SEQ - CH, CH)],
            sem_g.at[slot]).wait()

    def start_wb(b, slot):
        pltpu.async_copy(stage.at[slot], out_hbm.at[base + b, :, pl.ds(0, D)],
                         sem_w.at[slot])

    def wait_wb(b, slot):
        pltpu.make_async_copy(
            stage.at[slot], out_hbm.at[base + b, :, pl.ds(0, D)],
            sem_w.at[slot]).wait()

    start_gather(0, 0)
    wait_gather(0, 0)
    start_wb(0, 0)
    start_gather(1, 1)

    def row(b, carry):
        slot = b & 1
        other = 1 - slot
        wait_gather(b, slot)
        start_wb(b, slot)
        wait_wb(b - 1, other)
        start_gather(b + 1, other)
        return carry

    lax.fori_loop(1, BPW - 1, row, 0)

    wait_gather(BPW - 1, 1)
    start_wb(BPW - 1, 1)
    wait_wb(BPW - 2, 0)
    wait_wb(BPW - 1, 1)


def kernel(tokens, wte_weight, learned_embedding):
    tokens = tokens.astype(jnp.int32)
    idx3 = jnp.stack(
        [tokens[:, NT:NT + CH], tokens[:, SEQ - CH:SEQ]], axis=1)

    mesh = plsc.VectorSubcoreMesh(core_axis_name="c", subcore_axis_name="s")
    run = pl.kernel(
        _sc_body,
        out_type=jax.ShapeDtypeStruct((B, SEQ, 2 * D), jnp.float32),
        mesh=mesh,
        scratch_types=[
            pltpu.VMEM((BPW, 2, CH), jnp.int32),
            pltpu.VMEM((2, SEQ, D), jnp.float32),
            pltpu.SemaphoreType.DMA((2,)),
            pltpu.SemaphoreType.DMA((2,)),
        ],
        compiler_params=pltpu.CompilerParams(use_tc_tiling_on_sc=False),
    )
    x = run(idx3, wte_weight, learned_embedding)
    return x[:, :, :D]
